# baseline (speedup 1.0000x reference)
import jax
import jax.numpy as jnp
from jax.experimental import pallas as pl
from jax.experimental.pallas import tpu as pltpu

_HI = jax.lax.Precision.HIGHEST


def _stats(h):
    m = jnp.mean(h, axis=0, keepdims=True)
    sq = jnp.mean(h * h, axis=0, keepdims=True)
    return m, jax.lax.rsqrt(jnp.maximum(sq - m * m, 0.0) + 1e-5)


def _bn(h, g, b):
    m, isd = _stats(h)
    return (h - m) * isd * g + b


def _fwd_kernel(x_ref, adj_ref, bn_feat_g, bn_feat_b, W_feat, b_feat,
                bnc0_g, bnc0_b, Wc0, bc0,
                bnc1_g, bnc1_b, Wc1, bc1,
                bnc2_g, bnc2_b, Wc2, bc2,
                bnfc0_g, bnfc0_b, W_l0, b_l0,
                bn_h_g, bn_h_b, W_cls, b_cls,
                out_ref, h_ref):
    B, N, _ = adj_ref.shape

    dinv = jnp.concatenate(
        [jax.lax.rsqrt(jnp.sum(adj_ref[b], axis=1, keepdims=True) + 1.0)
         for b in range(B)], axis=0)

    x = x_ref[:]
    m, isd = _stats(x)
    srow = isd * bn_feat_g[:]
    Wp = W_feat[:] * srow.reshape(-1, 1)
    brow = (jnp.dot(bn_feat_b[:], W_feat[:],
                    preferred_element_type=jnp.float32, precision=_HI)
            + b_feat[:])
    h_ref[:, :] = jnp.maximum(
        jnp.dot(x - m, Wp, preferred_element_type=jnp.float32) + brow, 0.0)

    for (g, bb, W, bias) in ((bnc0_g, bnc0_b, Wc0, bc0),
                             (bnc1_g, bnc1_b, Wc1, bc1),
                             (bnc2_g, bnc2_b, Wc2, bc2)):
        h = h_ref[:, :]
        m, isd = _stats(h)
        srow = isd * g[:]
        Wp = W[:] * srow.reshape(-1, 1)
        brow = jnp.dot(bb[:], W[:],
                       preferred_element_type=jnp.float32, precision=_HI)
        hw = jnp.dot(h - m, Wp, preferred_element_type=jnp.float32) + brow
        v = dinv * hw
        for b in range(B):
            vb = v[b * N:(b + 1) * N]
            rb = jax.lax.dot_general(adj_ref[b], vb,
                                     (((0,), (0,)), ((), ())),
                                     preferred_element_type=jnp.float32)
            h_ref[b * N:(b + 1) * N, :] = jnp.maximum(
                dinv[b * N:(b + 1) * N] * (rb + vb) + bias[:], 0.0)

    pooled = jnp.concatenate(
        [jnp.mean(h_ref[b * N:(b + 1) * N, :], axis=0, keepdims=True)
         for b in range(B)], axis=0)

    z = _bn(pooled, bnfc0_g[:], bnfc0_b[:])
    z = jnp.maximum(
        jnp.dot(z, W_l0[:], preferred_element_type=jnp.float32,
                precision=_HI) + b_l0[:], 0.0)
    z = _bn(z, bn_h_g[:], bn_h_b[:])
    logits = jnp.dot(z, W_cls[:], preferred_element_type=jnp.float32,
                     precision=_HI) + b_cls[:]
    e = logits - jnp.max(logits, axis=1, keepdims=True)
    out_ref[:, :] = e - jnp.log(jnp.sum(jnp.exp(e), axis=1, keepdims=True))


def kernel(x, adj, bn_feat_g, bn_feat_b, W_feat, b_feat,
           bnc0_g, bnc0_b, Wc0, bc0,
           bnc1_g, bnc1_b, Wc1, bc1,
           bnc2_g, bnc2_b, Wc2, bc2,
           bnfc0_g, bnfc0_b, W_l0, b_l0,
           bn_h_g, bn_h_b, W_cls, b_cls):
    B, N, F = x.shape
    H = W_feat.shape[1]
    C = W_cls.shape[1]
    row = lambda a: a.reshape(1, -1)
    return pl.pallas_call(
        _fwd_kernel,
        out_shape=jax.ShapeDtypeStruct((B, C), jnp.float32),
        scratch_shapes=[pltpu.VMEM((B * N, H), jnp.float32)],
    )(x.reshape(B * N, F), adj,
      row(bn_feat_g), row(bn_feat_b), W_feat, row(b_feat),
      row(bnc0_g), row(bnc0_b), Wc0, row(bc0),
      row(bnc1_g), row(bnc1_b), Wc1, row(bc1),
      row(bnc2_g), row(bnc2_b), Wc2, row(bc2),
      row(bnfc0_g), row(bnfc0_b), W_l0, row(b_l0),
      row(bn_h_g), row(bn_h_b), W_cls, row(b_cls))
